# baseline (speedup 1.0000x reference)
import functools

import jax
import jax.numpy as jnp
from jax import lax
from jax.experimental import pallas as pl
from jax.experimental.pallas import tpu as pltpu
from jax.experimental.pallas import tpu_sc as plsc

_MASK_TOKEN_ID = 3
_MASK_RATIO = 0.3
_REGION_K = 14
_B = 1024
_N = 49
_H = 1024
_L = 20
_BB = 64
_LANES = 16
_SAMPLES_PER_WORKER = 32

_SCORES_T = jax.random.uniform(
    jax.random.fold_in(jax.random.key(0), 12345), (_B, _L)
).T


def _visual_kernel(attn_ref, img_ref, imgout_ref):
    a = attn_ref[...]
    ai = a[:, :, None]
    aj = a[:, None, :]
    idx = jax.lax.broadcasted_iota(jnp.int32, (1, _N, _N), 1)
    jdx = jax.lax.broadcasted_iota(jnp.int32, (1, _N, _N), 2)
    outranks = (aj > ai) | ((aj == ai) & (jdx < idx))
    vrank = outranks.astype(jnp.int32).sum(axis=2)
    keep = (vrank >= _REGION_K).astype(jnp.float32)
    row = jax.lax.broadcasted_iota(jnp.int32, (_BB, _BB), 0)
    col = jax.lax.broadcasted_iota(jnp.int32, (_BB, _BB), 1)
    eye = (row == col).astype(jnp.float32)
    keep_t = jax.lax.dot_general(
        keep, eye, (((0,), (0,)), ((), ())),
        preferred_element_type=jnp.float32,
    )
    imgout_ref[...] = img_ref[...] * keep_t[:, :, None]


def _run_visual(attn, img_t):
    grid = (_B // _BB,)
    return pl.pallas_call(
        _visual_kernel,
        grid=grid,
        in_specs=[
            pl.BlockSpec((_BB, _N), lambda b: (b, 0)),
            pl.BlockSpec((_N, _BB, _H), lambda b: (0, b, 0)),
        ],
        out_specs=pl.BlockSpec((_N, _BB, _H), lambda b: (0, b, 0)),
        out_shape=jax.ShapeDtypeStruct((_N, _B, _H), img_t.dtype),
        compiler_params=pltpu.CompilerParams(
            dimension_semantics=("arbitrary",),
        ),
    )(attn, img_t)


def _linguistic_body(q_hbm, s_hbm, out_hbm, q_v, s_v, o_v):
    wid = lax.axis_index("s") * 2 + lax.axis_index("c")
    base = wid * _SAMPLES_PER_WORKER
    pltpu.sync_copy(q_hbm.at[:, pl.ds(base, _SAMPLES_PER_WORKER)], q_v)
    pltpu.sync_copy(s_hbm.at[:, pl.ds(base, _SAMPLES_PER_WORKER)], s_v)

    ones = jnp.full((_LANES,), 1, jnp.int32)
    zeros = jnp.full((_LANES,), 0, jnp.int32)
    three = jnp.full((_LANES,), 3, jnp.int32)
    ten = jnp.full((_LANES,), 10, jnp.int32)

    for h in range(_SAMPLES_PER_WORKER // _LANES):
        sl = pl.ds(h * _LANES, _LANES)
        qs = [q_v[t, sl] for t in range(_L)]
        ss = [s_v[t, sl] for t in range(_L)]
        cint = [jnp.where(q > three, ones, zeros) for q in qs]
        ncont = cint[0]
        for t in range(1, _L):
            ncont = ncont + cint[t]
        nmask = jnp.maximum(ones, lax.div(ncont * jnp.full((_LANES,), 3, jnp.int32), ten))
        rank = [zeros] * _L
        for t in range(_L):
            for u in range(t):
                ci = jnp.where(ss[u] <= ss[t], ones, zeros)
                rank[t] = rank[t] + cint[u] * ci
                rank[u] = rank[u] + cint[t] * (ones - ci)
        anyc = jnp.where(ncont > 0, ones, zeros)
        for t in range(_L):
            doi = cint[t] * anyc * jnp.where(rank[t] < nmask, ones, zeros)
            o_v[t, sl] = jnp.where(doi > 0, three, qs[t])

    pltpu.sync_copy(o_v, out_hbm.at[:, pl.ds(base, _SAMPLES_PER_WORKER)])


def _run_linguistic(questions_t, scores_t):
    mesh = plsc.VectorSubcoreMesh(core_axis_name="c", subcore_axis_name="s")
    kern = functools.partial(
        pl.kernel,
        mesh=mesh,
        out_type=jax.ShapeDtypeStruct((_L, _B), questions_t.dtype),
        compiler_params=pltpu.CompilerParams(use_tc_tiling_on_sc=False),
        scratch_types=[
            pltpu.VMEM((_L, _SAMPLES_PER_WORKER), jnp.int32),
            pltpu.VMEM((_L, _SAMPLES_PER_WORKER), jnp.float32),
            pltpu.VMEM((_L, _SAMPLES_PER_WORKER), jnp.int32),
        ],
    )(_linguistic_body)
    return kern(questions_t, scores_t)


def kernel(questions, img_features, attn_weights):
    imgout_t = _run_visual(attn_weights, jnp.transpose(img_features, (1, 0, 2)))
    qout_t = _run_linguistic(questions.T, _SCORES_T)
    return (jnp.transpose(imgout_t, (1, 0, 2)), qout_t.T)


# speedup vs baseline: 1.1006x; 1.1006x over previous
import jax
import jax.numpy as jnp
from jax.experimental import pallas as pl
from jax.experimental.pallas import tpu as pltpu

_MASK_TOKEN_ID = 3
_MASK_RATIO = 0.3
_REGION_K = 14
_B = 1024
_N = 49
_H = 1024
_L = 20
_BB = 64


def _fused_kernel(q_ref, s_ref, attn_ref, img_ref, qout_ref, imgout_ref):
    a = attn_ref[...]
    ai = a[:, :, None]
    aj = a[:, None, :]
    idx = jax.lax.broadcasted_iota(jnp.int32, (1, _N, _N), 1)
    jdx = jax.lax.broadcasted_iota(jnp.int32, (1, _N, _N), 2)
    outranks = (aj > ai) | ((aj == ai) & (jdx < idx))
    vrank = outranks.astype(jnp.int32).sum(axis=2)
    keep = (vrank >= _REGION_K).astype(jnp.float32)
    row = jax.lax.broadcasted_iota(jnp.int32, (_BB, _BB), 0)
    col = jax.lax.broadcasted_iota(jnp.int32, (_BB, _BB), 1)
    eye = (row == col).astype(jnp.float32)
    keep_t = jax.lax.dot_general(
        keep, eye, ((( 0,), (0,)), ((), ())),
        preferred_element_type=jnp.float32,
    )
    imgout_ref[...] = img_ref[...] * keep_t[:, :, None]

    q = q_ref[...]
    s = s_ref[...]
    content = q > 3
    si = s[:, :, None]
    sj = s[:, None, :]
    li = jax.lax.broadcasted_iota(jnp.int32, (1, _L, _L), 1)
    lj = jax.lax.broadcasted_iota(jnp.int32, (1, _L, _L), 2)
    precedes = (sj < si) | ((sj == si) & (lj < li))
    precedes = precedes & content[:, None, :]
    lrank = precedes.astype(jnp.int32).sum(axis=2)
    n_content = content.astype(jnp.int32).sum(axis=1, keepdims=True)
    n_mask = jnp.maximum(
        1, jnp.floor(n_content.astype(jnp.float32) * _MASK_RATIO).astype(jnp.int32)
    )
    do_mask = content & (lrank < n_mask) & (n_content > 0)
    qout_ref[...] = jnp.where(do_mask, jnp.asarray(_MASK_TOKEN_ID, q.dtype), q)


def _run(questions, scores, attn, img_t):
    grid = (_B // _BB,)
    qout, imgout = pl.pallas_call(
        _fused_kernel,
        grid=grid,
        in_specs=[
            pl.BlockSpec((_BB, _L), lambda b: (b, 0)),
            pl.BlockSpec((_BB, _L), lambda b: (b, 0)),
            pl.BlockSpec((_BB, _N), lambda b: (b, 0)),
            pl.BlockSpec((_N, _BB, _H), lambda b: (0, b, 0)),
        ],
        out_specs=[
            pl.BlockSpec((_BB, _L), lambda b: (b, 0)),
            pl.BlockSpec((_N, _BB, _H), lambda b: (0, b, 0)),
        ],
        out_shape=[
            jax.ShapeDtypeStruct((_B, _L), questions.dtype),
            jax.ShapeDtypeStruct((_N, _B, _H), img_t.dtype),
        ],
        compiler_params=pltpu.CompilerParams(
            dimension_semantics=("arbitrary",),
        ),
    )(questions, scores, attn, img_t)
    return qout, imgout


_SCORES = jax.random.uniform(
    jax.random.fold_in(jax.random.key(0), 12345), (_B, _L)
)


def kernel(questions, img_features, attn_weights):
    scores = _SCORES
    qout, imgout_t = _run(
        questions,
        scores,
        attn_weights,
        jnp.transpose(img_features, (1, 0, 2)),
    )
    return (jnp.transpose(imgout_t, (1, 0, 2)), qout)
